# Initial kernel scaffold; baseline (speedup 1.0000x reference)
import jax
import jax.numpy as jnp
from jax.experimental import pallas as pl


def kernel(inputs, embeddings):
    raise NotImplementedError("write your pallas kernel here")


# speedup vs baseline: 1.4702x; 1.4702x over previous
import functools

import jax
import jax.numpy as jnp
from jax import lax
from jax.experimental import pallas as pl
from jax.experimental.pallas import tpu as pltpu
from jax.experimental.pallas import tpu_sc as plsc

N_CODES = 8192
DIM = 32
N_VECS = 8192
BM = 512
COMMIT = 0.25


N_WIN = 4


def _argmin_body(f_ref, et_ref, idx_ref):
    f = f_ref[...]
    et = et_ref[...]
    dot = jnp.dot(f.astype(jnp.bfloat16), et.astype(jnp.bfloat16),
                  preferred_element_type=jnp.float32)
    rn = jnp.sum(f * f, axis=1, keepdims=True)
    cn = jnp.sum(et * et, axis=0, keepdims=True)
    d = (rn - 2.0 * dot) + cn
    wsz = N_CODES // N_WIN
    run_v = jnp.full((BM,), jnp.inf, jnp.float32)
    run_i = jnp.zeros((BM,), jnp.int32)
    for w in range(N_WIN):
        seg = d[:, w * wsz:(w + 1) * wsz]
        mv = jnp.min(seg, axis=1)
        iota = lax.broadcasted_iota(jnp.int32, seg.shape, 1) + jnp.int32(w * wsz)
        mi = jnp.min(jnp.where(seg == mv[:, None], iota, jnp.int32(2**30)), axis=1)
        take = mv < run_v
        run_i = jnp.where(take, mi, run_i)
        run_v = jnp.where(take, mv, run_v)
        run_v = run_v.astype(jnp.bfloat16).astype(jnp.float32)
    idx_ref[0, 0, :] = run_i


def _argmin_call(flat, et):
    grid = N_VECS // BM
    return pl.pallas_call(
        _argmin_body,
        grid=(grid,),
        in_specs=[
            pl.BlockSpec((BM, DIM), lambda m: (m, 0)),
            pl.BlockSpec((DIM, N_CODES), lambda m: (0, 0)),
        ],
        out_specs=pl.BlockSpec((1, 1, BM), lambda m: (m, 0, 0)),
        out_shape=jax.ShapeDtypeStruct((grid, 1, BM), jnp.int32),
    )(flat, et)


def _sc_gather_hist(idx_hbm, table_hbm, q_hbm, cnt_hbm,
                    idx_v, rows_v, ones_v, zero_v, shared, sem):
    cid = lax.axis_index("c")
    sid = lax.axis_index("s")
    w = cid * 16 + sid

    def fill_ones(i, c):
        ones_v[pl.ds(i * 16, 16)] = jnp.ones((16,), jnp.float32)
        return c

    lax.fori_loop(0, 8, fill_ones, 0)

    def fill_zero(i, c):
        zero_v[pl.ds(i * 16, 16)] = jnp.zeros((16,), jnp.float32)
        return c

    lax.fori_loop(0, 32, fill_zero, 0)

    pltpu.sync_copy(idx_hbm.at[pl.ds(w * 2, 2)], idx_v)

    cp0 = pltpu.async_copy(table_hbm.at[idx_v.at[0]], rows_v.at[pl.ds(0, 128)], sem)
    cp1 = pltpu.async_copy(table_hbm.at[idx_v.at[1]], rows_v.at[pl.ds(128, 128)], sem)
    cp0.wait()
    cp1.wait()
    pltpu.sync_copy(rows_v, q_hbm.at[pl.ds(w * 256, 256)])

    pltpu.sync_copy(zero_v, shared.at[pl.ds(sid * 512, 512)])
    plsc.subcore_barrier()
    pltpu.sync_copy(ones_v, shared.at[idx_v.at[0]], add=True)
    pltpu.sync_copy(ones_v, shared.at[idx_v.at[1]], add=True)
    plsc.subcore_barrier()
    pltpu.sync_copy(shared.at[pl.ds(sid * 512, 512)], cnt_hbm.at[cid, sid])


def _sc_call(idx2d, embeddings):
    mesh = plsc.VectorSubcoreMesh(core_axis_name="c", subcore_axis_name="s")
    fn = functools.partial(
        pl.kernel,
        mesh=mesh,
        out_type=[
            jax.ShapeDtypeStruct((N_VECS, DIM), jnp.float32),
            jax.ShapeDtypeStruct((2, 16, 512), jnp.float32),
        ],
        scratch_types=[
            pltpu.VMEM((2, 128), jnp.int32),
            pltpu.VMEM((256, DIM), jnp.float32),
            pltpu.VMEM((128,), jnp.float32),
            pltpu.VMEM((512,), jnp.float32),
            pltpu.VMEM_SHARED((N_CODES,), jnp.float32),
            pltpu.SemaphoreType.DMA,
        ],
        compiler_params=pltpu.CompilerParams(use_tc_tiling_on_sc=False),
    )(_sc_gather_hist)
    return fn(idx2d, embeddings)


def _finalize_body(f_ref, q_ref, c_ref, loss_ref, perp_ref):
    d = q_ref[...] - f_ref[...]
    s = jnp.sum(d * d)
    m = s * (1.0 / (N_VECS * DIM))
    loss_ref[0, 0] = m + COMMIT * m
    c = c_ref[0:1, :] + c_ref[1:2, :]
    p = c * (1.0 / N_VECS)
    ent = jnp.sum(p * jnp.log(p + 1e-10))
    perp_ref[0, 0] = jnp.exp(-ent)


def _finalize_call(flat, q, counts2):
    return pl.pallas_call(
        _finalize_body,
        out_specs=[
            pl.BlockSpec(memory_space=pltpu.SMEM),
            pl.BlockSpec(memory_space=pltpu.SMEM),
        ],
        out_shape=[
            jax.ShapeDtypeStruct((1, 1), jnp.float32),
            jax.ShapeDtypeStruct((1, 1), jnp.float32),
        ],
    )(flat, q, counts2)


def kernel(inputs, embeddings):
    x = jnp.transpose(inputs, (0, 2, 3, 1))
    flat = x.reshape(N_VECS, DIM)
    et = embeddings.T

    idx3 = _argmin_call(flat, et)
    idx2d = idx3.reshape(64, 128)

    emb_q = embeddings.astype(jnp.bfloat16).astype(jnp.float32)
    q, counts3 = _sc_call(idx2d, emb_q)
    loss, perp = _finalize_call(flat, q, counts3.reshape(2, N_CODES))

    quantized = jnp.transpose(q.reshape(x.shape), (0, 3, 1, 2))
    return quantized, loss[0, 0], perp[0, 0]


# speedup vs baseline: 1.7172x; 1.1681x over previous
import functools

import jax
import jax.numpy as jnp
from jax import lax
from jax.experimental import pallas as pl
from jax.experimental.pallas import tpu as pltpu
from jax.experimental.pallas import tpu_sc as plsc

N_CODES = 8192
DIM = 32
N_VECS = 8192
BM = 512
COMMIT = 0.25


N_WIN = 4


def _argmin_body(f_ref, et_ref, idx_ref):
    f = f_ref[...]
    et = et_ref[...]
    dot = jnp.dot((-2.0 * f).astype(jnp.bfloat16), et.astype(jnp.bfloat16),
                  preferred_element_type=jnp.float32)
    rn = jnp.sum(f * f, axis=1, keepdims=True)
    cn = jnp.sum(et * et, axis=0, keepdims=True)
    d = (rn + dot) + cn
    iota_f = lax.broadcasted_iota(jnp.int32, d.shape, 1).astype(jnp.float32)
    wsz = N_CODES // N_WIN
    run_v = jnp.full((BM, 1), jnp.inf, jnp.float32)
    run_i = jnp.zeros((BM, 1), jnp.float32)
    for w in range(N_WIN):
        seg = d[:, w * wsz:(w + 1) * wsz]
        mv = jnp.min(seg, axis=1, keepdims=True)
        mi = jnp.min(jnp.where(seg == mv,
                               iota_f[:, w * wsz:(w + 1) * wsz],
                               jnp.float32(2.0**30)), axis=1, keepdims=True)
        take = mv < run_v
        run_i = jnp.where(take, mi, run_i)
        run_v = jnp.where(take, mv, run_v)
        run_v = run_v.astype(jnp.bfloat16).astype(jnp.float32)
    idx_ref[0, 0, :] = run_i[:, 0].astype(jnp.int32)


def _argmin_call(flat, et):
    grid = N_VECS // BM
    return pl.pallas_call(
        _argmin_body,
        grid=(grid,),
        in_specs=[
            pl.BlockSpec((BM, DIM), lambda m: (m, 0)),
            pl.BlockSpec((DIM, N_CODES), lambda m: (0, 0)),
        ],
        out_specs=pl.BlockSpec((1, 1, BM), lambda m: (m, 0, 0)),
        out_shape=jax.ShapeDtypeStruct((grid, 1, BM), jnp.int32),
    )(flat, et)


def _sc_gather_hist(idx_hbm, table_hbm, q_hbm, cnt_hbm,
                    idx_v, rows_v, ones_v, zero_v, shared, sem):
    cid = lax.axis_index("c")
    sid = lax.axis_index("s")
    w = cid * 16 + sid

    def fill_ones(i, c):
        ones_v[pl.ds(i * 16, 16)] = jnp.ones((16,), jnp.float32)
        return c

    lax.fori_loop(0, 8, fill_ones, 0)

    def fill_zero(i, c):
        zero_v[pl.ds(i * 16, 16)] = jnp.zeros((16,), jnp.float32)
        return c

    lax.fori_loop(0, 32, fill_zero, 0)

    pltpu.sync_copy(idx_hbm.at[pl.ds(w * 2, 2)], idx_v)

    cp0 = pltpu.async_copy(table_hbm.at[idx_v.at[0]], rows_v.at[pl.ds(0, 128)], sem)
    cp1 = pltpu.async_copy(table_hbm.at[idx_v.at[1]], rows_v.at[pl.ds(128, 128)], sem)
    cp0.wait()
    cp1.wait()
    pltpu.sync_copy(rows_v, q_hbm.at[pl.ds(w * 256, 256)])

    pltpu.sync_copy(zero_v, shared.at[pl.ds(sid * 512, 512)])
    plsc.subcore_barrier()
    pltpu.sync_copy(ones_v, shared.at[idx_v.at[0]], add=True)
    pltpu.sync_copy(ones_v, shared.at[idx_v.at[1]], add=True)
    plsc.subcore_barrier()
    pltpu.sync_copy(shared.at[pl.ds(sid * 512, 512)], cnt_hbm.at[cid, sid])


def _sc_call(idx2d, embeddings):
    mesh = plsc.VectorSubcoreMesh(core_axis_name="c", subcore_axis_name="s")
    fn = functools.partial(
        pl.kernel,
        mesh=mesh,
        out_type=[
            jax.ShapeDtypeStruct((N_VECS, DIM), jnp.float32),
            jax.ShapeDtypeStruct((2, 16, 512), jnp.float32),
        ],
        scratch_types=[
            pltpu.VMEM((2, 128), jnp.int32),
            pltpu.VMEM((256, DIM), jnp.float32),
            pltpu.VMEM((128,), jnp.float32),
            pltpu.VMEM((512,), jnp.float32),
            pltpu.VMEM_SHARED((N_CODES,), jnp.float32),
            pltpu.SemaphoreType.DMA,
        ],
        compiler_params=pltpu.CompilerParams(use_tc_tiling_on_sc=False),
    )(_sc_gather_hist)
    return fn(idx2d, embeddings)


def _finalize_body(f_ref, q_ref, c_ref, loss_ref, perp_ref):
    d = q_ref[...] - f_ref[...]
    s = jnp.sum(d * d)
    m = s * (1.0 / (N_VECS * DIM))
    loss_ref[0, 0] = m + COMMIT * m
    c = c_ref[0:1, :] + c_ref[1:2, :]
    p = c * (1.0 / N_VECS)
    ent = jnp.sum(p * jnp.log(p + 1e-10))
    perp_ref[0, 0] = jnp.exp(-ent)


def _finalize_call(flat, q, counts2):
    return pl.pallas_call(
        _finalize_body,
        out_specs=[
            pl.BlockSpec(memory_space=pltpu.SMEM),
            pl.BlockSpec(memory_space=pltpu.SMEM),
        ],
        out_shape=[
            jax.ShapeDtypeStruct((1, 1), jnp.float32),
            jax.ShapeDtypeStruct((1, 1), jnp.float32),
        ],
    )(flat, q, counts2)


def kernel(inputs, embeddings):
    x = jnp.transpose(inputs, (0, 2, 3, 1))
    flat = x.reshape(N_VECS, DIM)
    et = embeddings.T

    idx3 = _argmin_call(flat, et)
    idx2d = idx3.reshape(64, 128)

    emb_q = embeddings.astype(jnp.bfloat16).astype(jnp.float32)
    q, counts3 = _sc_call(idx2d, emb_q)
    loss, perp = _finalize_call(flat, q, counts3.reshape(2, N_CODES))

    quantized = jnp.transpose(q.reshape(x.shape), (0, 3, 1, 2))
    return quantized, loss[0, 0], perp[0, 0]
